# Initial kernel scaffold; baseline (speedup 1.0000x reference)
import jax
import jax.numpy as jnp
from jax.experimental import pallas as pl


def kernel(x, W):
    raise NotImplementedError("write your pallas kernel here")


# speedup vs baseline: 2.7027x; 2.7027x over previous
import jax
import jax.numpy as jnp
from jax.experimental import pallas as pl


def _router_kernel(x_ref, w_ref, gate_ref, probs_ref, idx_ref):
    T, E = gate_ref.shape
    logits = jnp.dot(x_ref[...], w_ref[...], preferred_element_type=jnp.float32)
    gate_ref[...] = logits
    iota = jax.lax.broadcasted_iota(jnp.int32, (T, E), 1)
    top1 = jnp.max(logits, axis=1, keepdims=True)
    i1 = jnp.min(jnp.where(logits == top1, iota, E), axis=1, keepdims=True)
    masked = jnp.where(iota == i1, -jnp.inf, logits)
    top2 = jnp.max(masked, axis=1, keepdims=True)
    i2 = jnp.min(jnp.where(masked == top2, iota, E), axis=1, keepdims=True)
    t = jnp.exp(top2 - top1)
    p1 = 1.0 / (1.0 + t)
    p2 = t / (1.0 + t)
    probs_ref[...] = jnp.where(iota == i1, p1, jnp.where(iota == i2, p2, 0.0))
    iota2 = jax.lax.broadcasted_iota(jnp.int32, idx_ref.shape, 1)
    idx_ref[...] = jnp.where(iota2 == 0, i1, i2)


def kernel(x, W):
    B, S, H = x.shape
    E = W.shape[0]
    K = 2
    N = B * S
    T = 1024
    xf = x.reshape(N, H)
    wt = W.T

    gate, probs, idx = pl.pallas_call(
        _router_kernel,
        grid=(N // T,),
        in_specs=[
            pl.BlockSpec((T, H), lambda i: (i, 0)),
            pl.BlockSpec((H, E), lambda i: (0, 0)),
        ],
        out_specs=[
            pl.BlockSpec((T, E), lambda i: (i, 0)),
            pl.BlockSpec((T, E), lambda i: (i, 0)),
            pl.BlockSpec((T, K), lambda i: (i, 0)),
        ],
        out_shape=[
            jax.ShapeDtypeStruct((N, E), jnp.float32),
            jax.ShapeDtypeStruct((N, E), jnp.float32),
            jax.ShapeDtypeStruct((N, K), jnp.int32),
        ],
    )(xf, wt)
    return probs.reshape(B, S, E), idx.reshape(B, S, K), gate


# speedup vs baseline: 2.7146x; 1.0044x over previous
import jax
import jax.numpy as jnp
from jax.experimental import pallas as pl
from jax.experimental.pallas import tpu as pltpu


def _router_kernel(x_ref, w_ref, gate_ref, probs_ref, idx_ref):
    T, E = gate_ref.shape
    logits = jnp.dot(x_ref[...], w_ref[...], preferred_element_type=jnp.float32)
    gate_ref[...] = logits
    iota = jax.lax.broadcasted_iota(jnp.int32, (T, E), 1)
    top1 = jnp.max(logits, axis=1, keepdims=True)
    i1 = jnp.min(jnp.where(logits == top1, iota, E), axis=1, keepdims=True)
    masked = jnp.where(iota == i1, -jnp.inf, logits)
    top2 = jnp.max(masked, axis=1, keepdims=True)
    i2 = jnp.min(jnp.where(masked == top2, iota, E), axis=1, keepdims=True)
    t = jnp.exp(top2 - top1)
    p1 = 1.0 / (1.0 + t)
    p2 = t / (1.0 + t)
    probs_ref[...] = jnp.where(iota == i1, p1, jnp.where(iota == i2, p2, 0.0))
    iota2 = jax.lax.broadcasted_iota(jnp.int32, idx_ref.shape, 1)
    idx_ref[...] = jnp.where(iota2 == 0, i1, i2)


def kernel(x, W):
    B, S, H = x.shape
    E = W.shape[0]
    K = 2
    N = B * S
    T = 1024
    xf = x.reshape(N, H)
    wt = W.T

    gate, probs, idx = pl.pallas_call(
        _router_kernel,
        grid=(N // T,),
        in_specs=[
            pl.BlockSpec((T, H), lambda i: (i, 0)),
            pl.BlockSpec((H, E), lambda i: (0, 0)),
        ],
        out_specs=[
            pl.BlockSpec((T, E), lambda i: (i, 0)),
            pl.BlockSpec((T, E), lambda i: (i, 0)),
            pl.BlockSpec((T, K), lambda i: (i, 0)),
        ],
        out_shape=[
            jax.ShapeDtypeStruct((N, E), jnp.float32),
            jax.ShapeDtypeStruct((N, E), jnp.float32),
            jax.ShapeDtypeStruct((N, K), jnp.int32),
        ],
        compiler_params=pltpu.CompilerParams(
            dimension_semantics=("parallel",),
        ),
    )(xf, wt)
    return probs.reshape(B, S, E), idx.reshape(B, S, K), gate


# speedup vs baseline: 2.7651x; 1.0186x over previous
import jax
import jax.numpy as jnp
from jax.experimental import pallas as pl
from jax.experimental.pallas import tpu as pltpu


def _router_kernel(x_ref, w_ref, gate_ref, probs_ref, idx_ref):
    T, E = gate_ref.shape
    logits = jnp.dot(x_ref[...], w_ref[...], preferred_element_type=jnp.float32)
    gate_ref[...] = logits
    iota = jax.lax.broadcasted_iota(jnp.int32, (T, E), 1)
    top1 = jnp.max(logits, axis=1, keepdims=True)
    i1 = jnp.min(jnp.where(logits == top1, iota, E), axis=1, keepdims=True)
    masked = jnp.where(iota == i1, -jnp.inf, logits)
    top2 = jnp.max(masked, axis=1, keepdims=True)
    i2 = jnp.min(jnp.where(masked == top2, iota, E), axis=1, keepdims=True)
    t = jnp.exp(top2 - top1)
    p1 = 1.0 / (1.0 + t)
    p2 = t / (1.0 + t)
    probs_ref[...] = jnp.where(iota == i1, p1, jnp.where(iota == i2, p2, 0.0))
    iota2 = jax.lax.broadcasted_iota(jnp.int32, idx_ref.shape, 1)
    idx_ref[...] = jnp.where(iota2 == 0, i1, i2)


def kernel(x, W):
    B, S, H = x.shape
    E = W.shape[0]
    K = 2
    N = B * S
    T = 2048
    xf = x.reshape(N, H)
    wt = W.T

    gate, probs, idx = pl.pallas_call(
        _router_kernel,
        grid=(N // T,),
        in_specs=[
            pl.BlockSpec((T, H), lambda i: (i, 0)),
            pl.BlockSpec((H, E), lambda i: (0, 0)),
        ],
        out_specs=[
            pl.BlockSpec((T, E), lambda i: (i, 0)),
            pl.BlockSpec((T, E), lambda i: (i, 0)),
            pl.BlockSpec((T, K), lambda i: (i, 0)),
        ],
        out_shape=[
            jax.ShapeDtypeStruct((N, E), jnp.float32),
            jax.ShapeDtypeStruct((N, E), jnp.float32),
            jax.ShapeDtypeStruct((N, K), jnp.int32),
        ],
        compiler_params=pltpu.CompilerParams(
            dimension_semantics=("parallel",),
        ),
    )(xf, wt)
    return probs.reshape(B, S, E), idx.reshape(B, S, K), gate
